# baseline (speedup 1.0000x reference)
import functools

import jax
import jax.numpy as jnp
from jax import lax
from jax.experimental import pallas as pl
from jax.experimental.pallas import tpu as pltpu
from jax.experimental.pallas import tpu_sc as plsc

VOCAB = 1000000
EMBED_DIM = 32
BATCH = 16384

NUM_CORES = 2
NUM_SUBCORES = 16
LANES = 16
NW = NUM_CORES * NUM_SUBCORES
BPW = BATCH // NW
IDX_CHUNK = 128
N_GATHERS = (2 * BPW) // IDX_CHUNK

MASK_LO = 65535
MASK_HI = -65536


def _sc_body(inp_hbm, table_hbm, out_hbm, inp_v, idx_v, rows_v, sem):
    wid = lax.axis_index("s") * NUM_CORES + lax.axis_index("c")
    base = wid * BPW

    pltpu.sync_copy(inp_hbm.at[pl.ds(base, BPW)], inp_v)

    for i in range(BPW // LANES):
        v = inp_v[pl.ds(i * LANES, LANES)]
        idx_v[pl.ds(i * LANES, LANES)] = v & MASK_HI
        idx_v[pl.ds(BPW + i * LANES, LANES)] = v & MASK_LO

    copies = []
    for j in range(N_GATHERS):
        copies.append(
            pltpu.async_copy(
                table_hbm.at[idx_v.at[pl.ds(j * IDX_CHUNK, IDX_CHUNK)]],
                rows_v.at[pl.ds(j * IDX_CHUNK, IDX_CHUNK)],
                sem,
            )
        )
    for c in copies:
        c.wait()

    def add_row(r, carry):
        a0 = rows_v[r, pl.ds(0, LANES)]
        b0 = rows_v[BPW + r, pl.ds(0, LANES)]
        rows_v[r, pl.ds(0, LANES)] = a0 + b0
        a1 = rows_v[r, pl.ds(LANES, LANES)]
        b1 = rows_v[BPW + r, pl.ds(LANES, LANES)]
        rows_v[r, pl.ds(LANES, LANES)] = a1 + b1
        return carry

    lax.fori_loop(0, BPW, add_row, 0)

    pltpu.sync_copy(rows_v.at[pl.ds(0, BPW)], out_hbm.at[pl.ds(base, BPW)])


@jax.jit
def _compositional_embedding(inputs, table):
    mesh = plsc.VectorSubcoreMesh(core_axis_name="c", subcore_axis_name="s")
    f = pl.kernel(
        _sc_body,
        out_type=jax.ShapeDtypeStruct((BATCH, EMBED_DIM), jnp.float32),
        mesh=mesh,
        scratch_types=[
            pltpu.VMEM((BPW,), jnp.int32),
            pltpu.VMEM((2 * BPW,), jnp.int32),
            pltpu.VMEM((2 * BPW, EMBED_DIM), jnp.float32),
            pltpu.SemaphoreType.DMA,
        ],
        compiler_params=pltpu.CompilerParams(use_tc_tiling_on_sc=False),
    )
    return f(inputs, table)


def kernel(inputs, table):
    return _compositional_embedding(inputs, table)


# speedup vs baseline: 1.0167x; 1.0167x over previous
import jax
import jax.numpy as jnp
from jax import lax
from jax.experimental import pallas as pl
from jax.experimental.pallas import tpu as pltpu
from jax.experimental.pallas import tpu_sc as plsc

VOCAB = 1000000
EMBED_DIM = 32
BATCH = 16384
PACK = 4
PHYS_ROWS = VOCAB // PACK
PHYS_COLS = PACK * EMBED_DIM

NUM_CORES = 2
NUM_SUBCORES = 16
LANES = 16
NW = NUM_CORES * NUM_SUBCORES
BPW = BATCH // NW
OPW = BPW // PACK
IDX_CHUNK = 128
N_GATHERS = BPW // IDX_CHUNK
N_HI = 16


def _sc_body(inp_hbm, table_hbm, out_hbm, inp_v, idx_v, hidx_v, lo_v, hi_v,
             out_v, sem):
    wid = lax.axis_index("s") * NUM_CORES + lax.axis_index("c")
    base = wid * BPW

    pltpu.sync_copy(inp_hbm.at[pl.ds(base, BPW)], inp_v)

    hidx_v[...] = lax.iota(jnp.int32, LANES) << 14
    hi_cp = pltpu.async_copy(table_hbm.at[hidx_v], hi_v, sem)

    for i in range(BPW // LANES):
        v = inp_v[pl.ds(i * LANES, LANES)]
        idx_v[pl.ds(i * LANES, LANES)] = (v & 65535) >> 2

    copies = []
    for j in range(N_GATHERS):
        copies.append(
            pltpu.async_copy(
                table_hbm.at[idx_v.at[pl.ds(j * IDX_CHUNK, IDX_CHUNK)]],
                lo_v.at[pl.ds(j * IDX_CHUNK, IDX_CHUNK)],
                sem,
            )
        )
    hi_cp.wait()
    for c in copies:
        c.wait()

    def combine(g, carry):
        v16 = inp_v[pl.ds(g * LANES, LANES)]
        for j in range(LANES):
            v = v16[j]
            r = g * LANES + j
            k = v >> 16
            sub = (v & 3) * EMBED_DIM
            orow = r >> 2
            ocol = (j & 3) * EMBED_DIM
            a0 = lo_v[r, pl.ds(sub, LANES)]
            b0 = hi_v[k, pl.ds(0, LANES)]
            out_v[orow, pl.ds(ocol, LANES)] = a0 + b0
            a1 = lo_v[r, pl.ds(sub + LANES, LANES)]
            b1 = hi_v[k, pl.ds(LANES, LANES)]
            out_v[orow, pl.ds(ocol + LANES, LANES)] = a1 + b1
        return carry

    lax.fori_loop(0, BPW // LANES, combine, 0)

    pltpu.sync_copy(out_v, out_hbm.at[pl.ds(wid * OPW, OPW)])


@jax.jit
def _compositional_embedding(inputs, table):
    table_p = jnp.reshape(table, (PHYS_ROWS, PHYS_COLS))
    mesh = plsc.VectorSubcoreMesh(core_axis_name="c", subcore_axis_name="s")
    f = pl.kernel(
        _sc_body,
        out_type=jax.ShapeDtypeStruct((BATCH // PACK, PHYS_COLS), jnp.float32),
        mesh=mesh,
        scratch_types=[
            pltpu.VMEM((BPW,), jnp.int32),
            pltpu.VMEM((BPW,), jnp.int32),
            pltpu.VMEM((LANES,), jnp.int32),
            pltpu.VMEM((BPW, PHYS_COLS), jnp.float32),
            pltpu.VMEM((N_HI, PHYS_COLS), jnp.float32),
            pltpu.VMEM((OPW, PHYS_COLS), jnp.float32),
            pltpu.SemaphoreType.DMA,
        ],
        compiler_params=pltpu.CompilerParams(use_tc_tiling_on_sc=True),
    )
    out_p = f(inputs, table_p)
    return jnp.reshape(out_p, (BATCH, EMBED_DIM))


def kernel(inputs, table):
    return _compositional_embedding(inputs, table)


# speedup vs baseline: 4.1425x; 4.0744x over previous
import jax
import jax.numpy as jnp
from jax import lax
from jax.experimental import pallas as pl
from jax.experimental.pallas import tpu as pltpu
from jax.experimental.pallas import tpu_sc as plsc

VOCAB = 1000000
EMBED_DIM = 32
BATCH = 16384
PACK = 4
PHYS_COLS = PACK * EMBED_DIM
LO_ROWS = 65536
N_HI = 16
LO_PHYS = LO_ROWS // PACK
HI_PHYS = N_HI // PACK

NUM_CORES = 2
NUM_SUBCORES = 16
LANES = 16
NW = NUM_CORES * NUM_SUBCORES
BPW = BATCH // NW
IDX_CHUNK = 128
N_GATHERS = BPW // IDX_CHUNK


def _sc_body(inp_hbm, staged_hbm, out_hbm, inp_v, idx_v, lo_v, hi_v, out_v,
             sem):
    wid = lax.axis_index("s") * NUM_CORES + lax.axis_index("c")
    base = wid * BPW

    pltpu.sync_copy(inp_hbm.at[pl.ds(base, BPW)], inp_v)
    hi_cp = pltpu.async_copy(staged_hbm.at[pl.ds(LO_PHYS, HI_PHYS)], hi_v, sem)

    for i in range(BPW // LANES):
        v = inp_v[pl.ds(i * LANES, LANES)]
        idx_v[pl.ds(i * LANES, LANES)] = (v & 65535) >> 2

    copies = []
    for j in range(N_GATHERS):
        copies.append(
            pltpu.async_copy(
                staged_hbm.at[idx_v.at[pl.ds(j * IDX_CHUNK, IDX_CHUNK)]],
                lo_v.at[pl.ds(j * IDX_CHUNK, IDX_CHUNK)],
                sem,
            )
        )
    hi_cp.wait()
    for c in copies:
        c.wait()

    iota = lax.iota(jnp.int32, LANES)

    def combine(eg, carry):
        e16 = inp_v[pl.ds(eg * LANES, LANES)]
        row_vec = iota + eg * LANES
        losub = (e16 & 3) * EMBED_DIM
        k = e16 >> 16
        hirow = k >> 2
        hisub = (k & 3) * EMBED_DIM
        for c in range(EMBED_DIM):
            lo_c = plsc.load_gather(lo_v, [row_vec, losub + c])
            hi_c = plsc.load_gather(hi_v, [hirow, hisub + c])
            out_v[c, pl.ds(eg * LANES, LANES)] = lo_c + hi_c
        return carry

    lax.fori_loop(0, BPW // LANES, combine, 0)

    pltpu.sync_copy(out_v, out_hbm.at[:, pl.ds(base, BPW)])


@jax.jit
def _compositional_embedding(inputs, table):
    lo_part = lax.slice(table, (0, 0), (LO_ROWS, EMBED_DIM))
    hi_part = lax.slice(
        table, (0, 0), ((N_HI - 1) * LO_ROWS + 1, EMBED_DIM),
        strides=(LO_ROWS, 1),
    )
    staged = jnp.reshape(
        jnp.concatenate([lo_part, hi_part], axis=0),
        (LO_PHYS + HI_PHYS, PHYS_COLS),
    )

    mesh = plsc.VectorSubcoreMesh(core_axis_name="c", subcore_axis_name="s")
    f = pl.kernel(
        _sc_body,
        out_type=jax.ShapeDtypeStruct((EMBED_DIM, BATCH), jnp.float32),
        mesh=mesh,
        scratch_types=[
            pltpu.VMEM((BPW,), jnp.int32),
            pltpu.VMEM((BPW,), jnp.int32),
            pltpu.VMEM((BPW, PHYS_COLS), jnp.float32),
            pltpu.VMEM((HI_PHYS, PHYS_COLS), jnp.float32),
            pltpu.VMEM((EMBED_DIM, BPW), jnp.float32),
            pltpu.SemaphoreType.DMA,
        ],
        compiler_params=pltpu.CompilerParams(
            use_tc_tiling_on_sc=True, needs_layout_passes=False
        ),
    )
    out_t = f(inputs, staged)
    return jnp.transpose(out_t)


def kernel(inputs, table):
    return _compositional_embedding(inputs, table)


# speedup vs baseline: 4.8212x; 1.1638x over previous
import jax
import jax.numpy as jnp
from jax import lax
from jax.experimental import pallas as pl
from jax.experimental.pallas import tpu as pltpu
from jax.experimental.pallas import tpu_sc as plsc

VOCAB = 1000000
EMBED_DIM = 32
BATCH = 16384
PACK = 4
PHYS_COLS = PACK * EMBED_DIM
LO_ROWS = 65536
N_HI = 16
LO_PHYS = LO_ROWS // PACK
HI_PHYS = N_HI // PACK

NUM_CORES = 2
NUM_SUBCORES = 16
LANES = 16
NW = NUM_CORES * NUM_SUBCORES
BPW = BATCH // NW
OUT_STRIDE = BPW + 1
IDX_CHUNK = 128
N_GATHERS = BPW // IDX_CHUNK


def _sc_body(inp_hbm, lo_hbm, hi_hbm, out_hbm, inp_v, idx_v, lo_v, hi_v,
             out_v, sem):
    wid = lax.axis_index("s") * NUM_CORES + lax.axis_index("c")
    base = wid * BPW

    pltpu.sync_copy(inp_hbm.at[pl.ds(base, BPW)], inp_v)
    hi_cp = pltpu.async_copy(hi_hbm, hi_v, sem)

    for i in range(BPW // LANES):
        v = inp_v[pl.ds(i * LANES, LANES)]
        idx_v[pl.ds(i * LANES, LANES)] = (v & 65535) >> 2

    copies = []
    for j in range(N_GATHERS):
        copies.append(
            pltpu.async_copy(
                lo_hbm.at[idx_v.at[pl.ds(j * IDX_CHUNK, IDX_CHUNK)]],
                lo_v.at[pl.ds(j * IDX_CHUNK, IDX_CHUNK)],
                sem,
            )
        )
    hi_cp.wait()
    for c in copies:
        c.wait()

    c_lo = lax.iota(jnp.int32, LANES)
    c_hi = c_lo + LANES

    def combine(g, carry):
        v16 = inp_v[pl.ds(g * LANES, LANES)]
        for j in range(LANES):
            v = v16[j]
            r = g * LANES + j
            k = v >> 16
            sub = (v & 3) * EMBED_DIM
            hirow = k >> 2
            hisub = (k & 3) * EMBED_DIM
            rvec = jnp.full((LANES,), r, jnp.int32)
            s0 = lo_v[r, pl.ds(sub, LANES)] + hi_v[hirow, pl.ds(hisub, LANES)]
            s1 = (lo_v[r, pl.ds(sub + LANES, LANES)]
                  + hi_v[hirow, pl.ds(hisub + LANES, LANES)])
            plsc.store_scatter(out_v, [c_lo, rvec], s0)
            plsc.store_scatter(out_v, [c_hi, rvec], s1)
        return carry

    lax.fori_loop(0, BPW // LANES, combine, 0)

    pltpu.sync_copy(out_v.at[:, pl.ds(0, BPW)], out_hbm.at[:, pl.ds(base, BPW)])


@jax.jit
def _compositional_embedding(inputs, table):
    lo_p = jnp.reshape(
        lax.slice(table, (0, 0), (LO_ROWS, EMBED_DIM)), (LO_PHYS, PHYS_COLS)
    )
    hi_p = jnp.reshape(
        lax.slice(table, (0, 0), ((N_HI - 1) * LO_ROWS + 1, EMBED_DIM),
                  strides=(LO_ROWS, 1)),
        (HI_PHYS, PHYS_COLS),
    )

    mesh = plsc.VectorSubcoreMesh(core_axis_name="c", subcore_axis_name="s")
    f = pl.kernel(
        _sc_body,
        out_type=jax.ShapeDtypeStruct((EMBED_DIM, BATCH), jnp.float32),
        mesh=mesh,
        scratch_types=[
            pltpu.VMEM((BPW,), jnp.int32),
            pltpu.VMEM((BPW,), jnp.int32),
            pltpu.VMEM((BPW, PHYS_COLS), jnp.float32),
            pltpu.VMEM((HI_PHYS, PHYS_COLS), jnp.float32),
            pltpu.VMEM((EMBED_DIM, OUT_STRIDE), jnp.float32),
            pltpu.SemaphoreType.DMA,
        ],
        compiler_params=pltpu.CompilerParams(
            use_tc_tiling_on_sc=True, needs_layout_passes=False
        ),
    )
    out_t = f(inputs, lo_p, hi_p)
    return jnp.transpose(out_t)


def kernel(inputs, table):
    return _compositional_embedding(inputs, table)


# speedup vs baseline: 5.1421x; 1.0666x over previous
import jax
import jax.numpy as jnp
from jax import lax
from jax.experimental import pallas as pl
from jax.experimental.pallas import tpu as pltpu
from jax.experimental.pallas import tpu_sc as plsc

VOCAB = 1000000
EMBED_DIM = 32
BATCH = 16384
LO_ROWS = 65536
N_HI = 16

NUM_CORES = 2
NUM_SUBCORES = 16
LANES = 16
NW = NUM_CORES * NUM_SUBCORES
BPW = BATCH // NW
OUT_STRIDE = BPW + 1
IDX_CHUNK = 128
N_GATHERS = BPW // IDX_CHUNK


def _sc_body(inp_hbm, lo_hbm, hi_hbm, out_hbm, inp_v, idx_v, lo_v, hi_v,
             out_v, sem):
    wid = lax.axis_index("s") * NUM_CORES + lax.axis_index("c")
    base = wid * BPW

    pltpu.sync_copy(inp_hbm.at[pl.ds(base, BPW)], inp_v)
    hi_cp = pltpu.async_copy(hi_hbm, hi_v, sem)

    for i in range(BPW // LANES):
        v = inp_v[pl.ds(i * LANES, LANES)]
        idx_v[pl.ds(i * LANES, LANES)] = v & 65535

    copies = []
    for j in range(N_GATHERS):
        copies.append(
            pltpu.async_copy(
                lo_hbm.at[idx_v.at[pl.ds(j * IDX_CHUNK, IDX_CHUNK)]],
                lo_v.at[pl.ds(j * IDX_CHUNK, IDX_CHUNK)],
                sem,
            )
        )
    hi_cp.wait()
    for c in copies:
        c.wait()

    c_lo = lax.iota(jnp.int32, LANES)
    c_hi = c_lo + LANES

    def combine(g, carry):
        v16 = inp_v[pl.ds(g * LANES, LANES)]
        for j in range(LANES):
            v = v16[j]
            r = g * LANES + j
            k = v >> 16
            rvec = jnp.full((LANES,), r, jnp.int32)
            s0 = lo_v[r, pl.ds(0, LANES)] + hi_v[k, pl.ds(0, LANES)]
            s1 = lo_v[r, pl.ds(LANES, LANES)] + hi_v[k, pl.ds(LANES, LANES)]
            plsc.store_scatter(out_v, [c_lo, rvec], s0)
            plsc.store_scatter(out_v, [c_hi, rvec], s1)
        return carry

    lax.fori_loop(0, BPW // LANES, combine, 0)

    pltpu.sync_copy(out_v.at[:, pl.ds(0, BPW)], out_hbm.at[:, pl.ds(base, BPW)])


@jax.jit
def _compositional_embedding(inputs, table):
    lo_p = lax.slice(table, (0, 0), (LO_ROWS, EMBED_DIM))
    hi_p = lax.slice(table, (0, 0), ((N_HI - 1) * LO_ROWS + 1, EMBED_DIM),
                     strides=(LO_ROWS, 1))

    mesh = plsc.VectorSubcoreMesh(core_axis_name="c", subcore_axis_name="s")
    f = pl.kernel(
        _sc_body,
        out_type=jax.ShapeDtypeStruct((EMBED_DIM, BATCH), jnp.float32),
        mesh=mesh,
        scratch_types=[
            pltpu.VMEM((BPW,), jnp.int32),
            pltpu.VMEM((BPW,), jnp.int32),
            pltpu.VMEM((BPW, EMBED_DIM), jnp.float32),
            pltpu.VMEM((N_HI, EMBED_DIM), jnp.float32),
            pltpu.VMEM((EMBED_DIM, OUT_STRIDE), jnp.float32),
            pltpu.SemaphoreType.DMA,
        ],
        compiler_params=pltpu.CompilerParams(
            use_tc_tiling_on_sc=False, needs_layout_passes=False
        ),
    )
    out_t = f(inputs, lo_p, hi_p)
    return jnp.transpose(out_t)


def kernel(inputs, table):
    return _compositional_embedding(inputs, table)


# speedup vs baseline: 6.1741x; 1.2007x over previous
import jax
import jax.numpy as jnp
from jax import lax
from jax.experimental import pallas as pl
from jax.experimental.pallas import tpu as pltpu
from jax.experimental.pallas import tpu_sc as plsc

VOCAB = 1000000
EMBED_DIM = 32
BATCH = 16384
LO_ROWS = 65536
N_HI = 16

NUM_CORES = 2
NUM_SUBCORES = 16
LANES = 16
NW = NUM_CORES * NUM_SUBCORES
UNROLL = 4


def _sc_body(inp_hbm, lo_hbm, hi_hbm, out_hbm, ids_v, col_v, hi_v, out_v,
             sem):
    d = lax.axis_index("s") * NUM_CORES + lax.axis_index("c")

    ids_cp = pltpu.async_copy(inp_hbm, ids_v, sem)
    col_cp = pltpu.async_copy(lo_hbm.at[d], col_v, sem)
    hi_cp = pltpu.async_copy(hi_hbm.at[d], hi_v, sem)
    ids_cp.wait()
    col_cp.wait()
    hi_cp.wait()

    def serve(g, carry):
        for j in range(UNROLL):
            off = (g * UNROLL + j) * LANES
            ids = ids_v[pl.ds(off, LANES)]
            lo = ids & 65535
            k = ids >> 16
            out_v[pl.ds(off, LANES)] = (
                plsc.load_gather(col_v, [lo]) + plsc.load_gather(hi_v, [k])
            )
        return carry

    lax.fori_loop(0, BATCH // (LANES * UNROLL), serve, 0)

    pltpu.sync_copy(out_v, out_hbm.at[d])


@jax.jit
def _compositional_embedding(inputs, table):
    tbl_t = jnp.transpose(table)
    lo_t = lax.slice(tbl_t, (0, 0), (EMBED_DIM, LO_ROWS))
    hi_t = lax.slice(tbl_t, (0, 0), (EMBED_DIM, (N_HI - 1) * LO_ROWS + 1),
                     strides=(1, LO_ROWS))

    mesh = plsc.VectorSubcoreMesh(core_axis_name="c", subcore_axis_name="s")
    f = pl.kernel(
        _sc_body,
        out_type=jax.ShapeDtypeStruct((EMBED_DIM, BATCH), jnp.float32),
        mesh=mesh,
        scratch_types=[
            pltpu.VMEM((BATCH,), jnp.int32),
            pltpu.VMEM((LO_ROWS,), jnp.float32),
            pltpu.VMEM((N_HI,), jnp.float32),
            pltpu.VMEM((BATCH,), jnp.float32),
            pltpu.SemaphoreType.DMA,
        ],
        compiler_params=pltpu.CompilerParams(
            use_tc_tiling_on_sc=False, needs_layout_passes=False
        ),
    )
    out_t = f(inputs, lo_t, hi_t)
    return jnp.transpose(out_t)


def kernel(inputs, table):
    return _compositional_embedding(inputs, table)


# speedup vs baseline: 8.3145x; 1.3467x over previous
import jax
import jax.numpy as jnp
from jax import lax
from jax.experimental import pallas as pl
from jax.experimental.pallas import tpu as pltpu
from jax.experimental.pallas import tpu_sc as plsc

VOCAB = 1000000
EMBED_DIM = 32
BATCH = 16384
LO_ROWS = 65536
N_HI = 16

NUM_CORES = 2
NUM_SUBCORES = 16
LANES = 16
NW = NUM_CORES * NUM_SUBCORES
UNROLL = 4


def _sc_body(inp_hbm, lo_hbm, hi_hbm, out_hbm, ids_v, col_v, hi_v, out_v,
             sem):
    d = lax.axis_index("s") * NUM_CORES + lax.axis_index("c")

    ids_cp = pltpu.async_copy(inp_hbm, ids_v, sem)
    col_cp = pltpu.async_copy(lo_hbm.at[d], col_v, sem)
    hi_cp = pltpu.async_copy(hi_hbm.at[d], hi_v, sem)
    ids_cp.wait()
    col_cp.wait()
    hi_cp.wait()

    def serve(g, carry):
        for j in range(UNROLL):
            off = (g * UNROLL + j) * LANES
            ids = ids_v[pl.ds(off, LANES)]
            lo = ids & 65535
            k = (ids >> 16) << 7
            out_v[pl.ds(off, LANES)] = (
                plsc.load_gather(col_v, [lo]) + plsc.load_gather(hi_v, [k])
            )
        return carry

    lax.fori_loop(0, BATCH // (LANES * UNROLL), serve, 0)

    pltpu.sync_copy(out_v, out_hbm.at[d])


@jax.jit
def _compositional_embedding(inputs, table):
    tbl_t = jnp.transpose(table)
    lo_t = lax.slice(tbl_t, (0, 0), (EMBED_DIM, LO_ROWS))
    hi_t = jnp.concatenate(
        [lax.slice(tbl_t, (0, k * LO_ROWS), (EMBED_DIM, k * LO_ROWS + 128))
         for k in range(N_HI)], axis=1)

    mesh = plsc.VectorSubcoreMesh(core_axis_name="c", subcore_axis_name="s")
    f = pl.kernel(
        _sc_body,
        out_type=jax.ShapeDtypeStruct((EMBED_DIM, BATCH), jnp.float32),
        mesh=mesh,
        scratch_types=[
            pltpu.VMEM((BATCH,), jnp.int32),
            pltpu.VMEM((LO_ROWS,), jnp.float32),
            pltpu.VMEM((N_HI * 128,), jnp.float32),
            pltpu.VMEM((BATCH,), jnp.float32),
            pltpu.SemaphoreType.DMA,
        ],
        compiler_params=pltpu.CompilerParams(
            use_tc_tiling_on_sc=False, needs_layout_passes=False
        ),
    )
    out_t = f(inputs, lo_t, hi_t)
    return jnp.transpose(out_t)


def kernel(inputs, table):
    return _compositional_embedding(inputs, table)


# speedup vs baseline: 9.6364x; 1.1590x over previous
import jax
import jax.numpy as jnp
from jax import lax
from jax.experimental import pallas as pl
from jax.experimental.pallas import tpu as pltpu
from jax.experimental.pallas import tpu_sc as plsc

VOCAB = 1000000
EMBED_DIM = 32
BATCH = 16384
LO_ROWS = 65536
N_HI = 16

NUM_CORES = 2
NUM_SUBCORES = 16
LANES = 16
NW = NUM_CORES * NUM_SUBCORES
UNROLL = 4


def _sc_body(inp_hbm, lo_hbm, hi_hbm, out_hbm, ids_v, col_v, hi_v, hi16_v,
             out_v, sem):
    d = lax.axis_index("s") * NUM_CORES + lax.axis_index("c")

    ids_cp = pltpu.async_copy(inp_hbm, ids_v, sem)
    col_cp = pltpu.async_copy(lo_hbm.at[d], col_v, sem)
    hi_cp = pltpu.async_copy(hi_hbm.at[d], hi_v, sem)
    ids_cp.wait()
    col_cp.wait()
    hi_cp.wait()

    iota = lax.iota(jnp.int32, LANES)
    hi16_v[...] = plsc.load_gather(hi_v, [iota * 128])

    def serve(g, carry):
        for j in range(UNROLL):
            off = (g * UNROLL + j) * LANES
            ids = ids_v[pl.ds(off, LANES)]
            lo = ids & 65535
            k = ids >> 16
            out_v[pl.ds(off, LANES)] = (
                plsc.load_gather(col_v, [lo]) + plsc.load_gather(hi16_v, [k])
            )
        return carry

    lax.fori_loop(0, BATCH // (LANES * UNROLL), serve, 0)

    pltpu.sync_copy(out_v, out_hbm.at[d])


@jax.jit
def _compositional_embedding(inputs, table):
    tbl_t = jnp.transpose(table)
    lo_t = lax.slice(tbl_t, (0, 0), (EMBED_DIM, LO_ROWS))
    hi_t = jnp.concatenate(
        [lax.slice(tbl_t, (0, k * LO_ROWS), (EMBED_DIM, k * LO_ROWS + 128))
         for k in range(N_HI)], axis=1)

    mesh = plsc.VectorSubcoreMesh(core_axis_name="c", subcore_axis_name="s")
    f = pl.kernel(
        _sc_body,
        out_type=jax.ShapeDtypeStruct((EMBED_DIM, BATCH), jnp.float32),
        mesh=mesh,
        scratch_types=[
            pltpu.VMEM((BATCH,), jnp.int32),
            pltpu.VMEM((LO_ROWS,), jnp.float32),
            pltpu.VMEM((N_HI * 128,), jnp.float32),
            pltpu.VMEM((LANES,), jnp.float32),
            pltpu.VMEM((BATCH,), jnp.float32),
            pltpu.SemaphoreType.DMA,
        ],
        compiler_params=pltpu.CompilerParams(
            use_tc_tiling_on_sc=False, needs_layout_passes=False
        ),
    )
    out_t = f(inputs, lo_t, hi_t)
    return jnp.transpose(out_t)


def kernel(inputs, table):
    return _compositional_embedding(inputs, table)


# speedup vs baseline: 10.6251x; 1.1026x over previous
import jax
import jax.numpy as jnp
from jax import lax
from jax.experimental import pallas as pl
from jax.experimental.pallas import tpu as pltpu
from jax.experimental.pallas import tpu_sc as plsc

VOCAB = 1000000
EMBED_DIM = 32
BATCH = 16384
LO_ROWS = 65536
N_HI = 16

NUM_CORES = 2
NUM_SUBCORES = 16
LANES = 16
NW = NUM_CORES * NUM_SUBCORES
UNROLL = 4


def _sc_body(inp_hbm, lo_hbm, hi_hbm, out_hbm, ids_v, col_v, hi16_v, out_v,
             sem):
    d = lax.axis_index("s") * NUM_CORES + lax.axis_index("c")

    ids_cp = pltpu.async_copy(inp_hbm, ids_v, sem)
    col_cp = pltpu.async_copy(lo_hbm.at[d], col_v, sem)
    hi_cp = pltpu.async_copy(hi_hbm.at[d], hi16_v, sem)
    ids_cp.wait()
    col_cp.wait()
    hi_cp.wait()

    def serve(g, carry):
        for j in range(UNROLL):
            off = (g * UNROLL + j) * LANES
            ids = ids_v[pl.ds(off, LANES)]
            lo = ids & 65535
            k = ids >> 16
            out_v[pl.ds(off, LANES)] = (
                plsc.load_gather(col_v, [lo]) + plsc.load_gather(hi16_v, [k])
            )
        return carry

    lax.fori_loop(0, BATCH // (LANES * UNROLL), serve, 0)

    pltpu.sync_copy(out_v, out_hbm.at[d])


@jax.jit
def _compositional_embedding(inputs, table):
    tbl_t = jnp.transpose(table)
    lo_t = lax.slice(tbl_t, (0, 0), (EMBED_DIM, LO_ROWS))
    hi_t = jnp.take(
        tbl_t, jnp.arange(0, N_HI * LO_ROWS, LO_ROWS, dtype=jnp.int32),
        axis=1)

    mesh = plsc.VectorSubcoreMesh(core_axis_name="c", subcore_axis_name="s")
    f = pl.kernel(
        _sc_body,
        out_type=jax.ShapeDtypeStruct((EMBED_DIM, BATCH), jnp.float32),
        mesh=mesh,
        scratch_types=[
            pltpu.VMEM((BATCH,), jnp.int32),
            pltpu.VMEM((LO_ROWS,), jnp.float32),
            pltpu.VMEM((N_HI,), jnp.float32),
            pltpu.VMEM((BATCH,), jnp.float32),
            pltpu.SemaphoreType.DMA,
        ],
        compiler_params=pltpu.CompilerParams(
            use_tc_tiling_on_sc=False, needs_layout_passes=False
        ),
    )
    out_t = f(inputs, lo_t, hi_t)
    return jnp.transpose(out_t)


def kernel(inputs, table):
    return _compositional_embedding(inputs, table)


# speedup vs baseline: 12.7253x; 1.1977x over previous
import jax
import jax.numpy as jnp
from jax import lax
from jax.experimental import pallas as pl
from jax.experimental.pallas import tpu as pltpu
from jax.experimental.pallas import tpu_sc as plsc

VOCAB = 1000000
EMBED_DIM = 32
BATCH = 16384
LO_ROWS = 65536
N_HI = 16

NUM_CORES = 2
NUM_SUBCORES = 16
LANES = 16
NW = NUM_CORES * NUM_SUBCORES
UNROLL = 4


def _sc_body(inp_hbm, lo_hbm, hi_hbm, out_hbm, ids_v, col_v, hi16_v, out_v,
             sem):
    d = lax.axis_index("s") * NUM_CORES + lax.axis_index("c")

    ids_cp = pltpu.async_copy(inp_hbm, ids_v, sem)
    col_cp = pltpu.async_copy(lo_hbm.at[d >> 3, :, d & 7], col_v, sem)
    hi_cp = pltpu.async_copy(hi_hbm.at[d], hi16_v, sem)
    ids_cp.wait()
    col_cp.wait()
    hi_cp.wait()

    def serve(g, carry):
        for j in range(UNROLL):
            off = (g * UNROLL + j) * LANES
            ids = ids_v[pl.ds(off, LANES)]
            lo = ids & 65535
            k = ids >> 16
            out_v[pl.ds(off, LANES)] = (
                plsc.load_gather(col_v, [lo >> 7, lo & 127])
                + plsc.load_gather(hi16_v, [k])
            )
        return carry

    lax.fori_loop(0, BATCH // (LANES * UNROLL), serve, 0)

    pltpu.sync_copy(out_v, out_hbm.at[d])


@jax.jit
def _compositional_embedding(inputs, table):
    tbl_t = jnp.transpose(table)
    lo_t = lax.slice(tbl_t, (0, 0), (EMBED_DIM, LO_ROWS))
    lo4 = jnp.transpose(jnp.reshape(lo_t, (4, 8, 512, 128)), (0, 2, 1, 3))
    hi_t = jnp.take(
        tbl_t, jnp.arange(0, N_HI * LO_ROWS, LO_ROWS, dtype=jnp.int32),
        axis=1)

    mesh = plsc.VectorSubcoreMesh(core_axis_name="c", subcore_axis_name="s")
    f = pl.kernel(
        _sc_body,
        out_type=jax.ShapeDtypeStruct((EMBED_DIM, BATCH), jnp.float32),
        mesh=mesh,
        scratch_types=[
            pltpu.VMEM((BATCH,), jnp.int32),
            pltpu.VMEM((LO_ROWS // 128, 128), jnp.float32),
            pltpu.VMEM((N_HI,), jnp.float32),
            pltpu.VMEM((BATCH,), jnp.float32),
            pltpu.SemaphoreType.DMA,
        ],
        compiler_params=pltpu.CompilerParams(
            use_tc_tiling_on_sc=False, needs_layout_passes=False
        ),
    )
    out_t = f(inputs, lo4, hi_t)
    return jnp.transpose(out_t)


def kernel(inputs, table):
    return _compositional_embedding(inputs, table)
